# baseline (speedup 1.0000x reference)
import functools

import jax
import jax.numpy as jnp
from jax import lax
from jax.experimental import pallas as pl
from jax.experimental.pallas import tpu as pltpu
from jax.experimental.pallas import tpu_sc as plsc

V = 512
E = 4096
F = V + E
VB = 512
NB = F // VB
NEB = E // VB
FX = 128
FC = 64

i32 = jnp.int32
f32 = jnp.float32


def _one_i(meta_ref, eb):
    cnt = meta_ref[0, :]
    off = meta_ref[1, :]
    e_row = (lax.broadcasted_iota(i32, (VB, V), 0) + eb * VB).astype(f32)
    one_i = jnp.where(
        (e_row > off[None, :] - 0.5) & (e_row < (off + cnt)[None, :] - 0.5),
        1.0, 0.0)
    return one_i, e_row, off


def _build_m(rank_ref, meta_ref, eb):
    one_i, e_row, off = _one_i(meta_ref, eb)
    rank_rows = jnp.floor(
        jnp.dot(one_i, rank_ref[...], preferred_element_type=f32) + 0.5)
    u_rows = rank_rows - jnp.concatenate(
        [jnp.zeros((VB, 1), f32), rank_rows[:, : V - 1]], axis=1)
    oe = jnp.sum(one_i * off[None, :], axis=1, keepdims=True)
    tgt = e_row[:, 0:1] - oe + 1.0
    one_j = jnp.where((u_rows > 0.5) & (jnp.abs(rank_rows - tgt) < 0.5),
                      1.0, 0.0)
    return one_i + one_j


def _tc_body(a_ref, x_ref, cf_ref, s_ref,
             x_out, c_out, s_out, a_out,
             m_st, rank_st, meta_st):
    bi = pl.program_id(0)
    bj = pl.program_id(1)

    @pl.when((bi == 0) & (bj == 0))
    def _corner():
        a = a_ref[0]
        row = lax.broadcasted_iota(i32, (V, V), 0)
        col = lax.broadcasted_iota(i32, (V, V), 1)
        u = jnp.where((a != 0) & (col > row), 1.0, 0.0).astype(f32)
        ut = jnp.where(row <= col, 1.0, 0.0).astype(f32)
        rank = jnp.floor(jnp.dot(u, ut, preferred_element_type=f32) + 0.5)
        cnt = jnp.sum(u, axis=1)
        sl = jnp.where(row < col, 1.0, 0.0).astype(f32)
        off = jnp.floor(jnp.sum(cnt[:, None] * sl, axis=0) + 0.5)
        rank_st[...] = rank
        meta_st[...] = jnp.concatenate(
            [cnt[None], off[None], jnp.zeros((6, V), f32)], axis=0)
        a_out[0] = jnp.zeros((VB, VB), f32)
        x_out[0] = x_ref[0]
        c_out[0] = cf_ref[0]
        s_out[0] = s_ref[0]

    @pl.when((bi == 0) & (bj > 0))
    def _top_band():
        eb = bj - 1
        m = _build_m(rank_st, meta_st, eb)
        a_out[0] = jnp.transpose(m)
        m_st[pl.ds(eb, 1)] = m[None]

    @pl.when((bi > 0) & (bj == 0))
    def _left_band():
        eb = bi - 1
        m = m_st[pl.ds(eb, 1)][0]
        a_out[0] = m
        x_out[0] = 0.5 * jnp.dot(m, x_ref[0], preferred_element_type=f32)
        c_out[0] = 0.5 * jnp.dot(m, cf_ref[0], preferred_element_type=f32)
        s_out[0] = 0.5 * jnp.dot(m, s_ref[0], preferred_element_type=f32)

    @pl.when((bi > 0) & (bj > 0))
    def _bulk_zero():
        a_out[0] = jnp.zeros((VB, VB), f32)


def _tc_call(A, x_prev, c_prev, s_prev):
    const = lambda i, j: (0, 0, 0)
    rowblk = lambda i, j: (0, i, 0)
    return pl.pallas_call(
        _tc_body,
        grid=(NB, NB),
        in_specs=[
            pl.BlockSpec((1, V, V), const),
            pl.BlockSpec((1, V, FX), const),
            pl.BlockSpec((1, V, FC), const),
            pl.BlockSpec((1, V, FX), const),
        ],
        out_specs=[
            pl.BlockSpec((1, VB, FX), rowblk),
            pl.BlockSpec((1, VB, FC), rowblk),
            pl.BlockSpec((1, VB, FX), rowblk),
            pl.BlockSpec((1, VB, VB), lambda i, j: (0, i, j)),
        ],
        out_shape=[
            jax.ShapeDtypeStruct((1, F, FX), f32),
            jax.ShapeDtypeStruct((1, F, FC), f32),
            jax.ShapeDtypeStruct((1, F, FX), f32),
            jax.ShapeDtypeStruct((1, F, F), f32),
        ],
        scratch_shapes=[
            pltpu.VMEM((NEB, VB, VB), f32),
            pltpu.VMEM((V, V), f32),
            pltpu.VMEM((8, V), f32),
        ],
    )(A, x_prev, c_prev, s_prev)


LANES = 16
NROW = 16
CAP = E + LANES


def _iota16():
    return lax.iota(i32, LANES)


def _vbroadcast(ref, idx_scalar):
    return plsc.load_gather(ref, [jnp.full((LANES,), idx_scalar, i32)])


def _sc_body(a_hbm, pid_hbm, out_hbm,
             sh_rank, sh_cnt, sh_poly,
             cntv, offv, polyv, rowbuf):
    c = lax.axis_index("c")
    s = lax.axis_index("s")
    w = s * 2 + c
    i0 = w * NROW

    def phase_a(abuf, rkbuf, colp, cnt8, poly16):
        for b in range(4):
            r0 = s * 32 + b * 8
            pltpu.sync_copy(a_hbm.at[0, pl.ds(r0, 8), :], abuf)

            def row_body(r, _):
                def ch_body(ch, run):
                    a = abuf[r, pl.ds(ch * LANES, LANES)]
                    cols = ch * LANES + _iota16()
                    u = jnp.where((a != 0) & (cols > (r0 + r)), 1, 0)
                    rkbuf[r, pl.ds(ch * LANES, LANES)] = plsc.cumsum(u) + run
                    return run + jnp.sum(u)
                cnt_r = lax.fori_loop(0, 32, ch_body, jnp.int32(0))
                plsc.store_scatter(cnt8, [jnp.full((LANES,), r, i32)],
                                   jnp.full((LANES,), cnt_r, i32),
                                   mask=_iota16() == 0)
                return 0
            lax.fori_loop(0, 8, row_body, 0)
            pltpu.sync_copy(rkbuf, sh_rank.at[pl.ds(r0, 8), :])
            pltpu.sync_copy(cnt8, sh_cnt.at[pl.ds(r0, 8)])

        for b in range(2):
            r0 = s * 32 + b * 16
            pltpu.sync_copy(pid_hbm.at[0, :, pl.ds(r0, 16)], colp)

            def pmax_body(k, acc):
                return jnp.maximum(acc, colp[k])
            pmx = lax.fori_loop(0, V, pmax_body,
                                jnp.full((LANES,), -(2 ** 30), i32))
            poly16[...] = pmx
            pltpu.sync_copy(poly16, sh_poly.at[pl.ds(r0, 16)])

    pl.run_scoped(
        phase_a,
        pltpu.VMEM((8, V), i32),
        pltpu.VMEM((8, V), i32),
        pltpu.VMEM((V, 16), i32),
        pltpu.VMEM((8,), i32),
        pltpu.VMEM((16,), i32),
    )
    plsc.subcore_barrier()

    pltpu.sync_copy(sh_cnt, cntv)
    pltpu.sync_copy(sh_poly, polyv)

    def off_body(t, tot):
        cv = cntv[pl.ds(t * LANES, LANES)]
        offv[pl.ds(t * LANES, LANES)] = plsc.cumsum(cv) - cv + tot
        return tot + jnp.sum(cv)
    lax.fori_loop(0, V // LANES, off_body, jnp.int32(0))

    def phase_bc(rowsa, cola, colr, plist):
        pltpu.sync_copy(a_hbm.at[0, pl.ds(i0, NROW), :], rowsa)
        pltpu.sync_copy(a_hbm.at[0, :, pl.ds(i0, NROW)], cola)
        pltpu.sync_copy(sh_rank.at[:, pl.ds(i0, NROW)], colr)

        rb2 = rowbuf

        def z_body(k, _):
            l = k // (F // (8 * LANES))
            base = (k % (F // (8 * LANES))) * 8 * LANES
            for q in range(8):
                rb2[l, pl.ds(base + q * LANES, LANES)] = jnp.zeros((LANES,), f32)
            return 0
        lax.fori_loop(0, NROW * F // (8 * LANES), z_body, 0)

        def top_scatter(scale):
            def row_body(l, _):
                i = i0 + l
                poly_i = _vbroadcast(polyv, i).astype(f32) * scale
                off_i = _vbroadcast(offv, i)

                def ch_body(ch, run):
                    a = rowsa[l, pl.ds(ch * LANES, LANES)]
                    cols = ch * LANES + _iota16()
                    u = (a != 0) & (cols > i)
                    ui = jnp.where(u, 1, 0)
                    rk = plsc.cumsum(ui) + run
                    e = off_i + rk - 1
                    plsc.store_scatter(rb2, [jnp.full((LANES,), l, i32), V + e],
                                       poly_i, mask=u)
                    return run + jnp.sum(ui)
                lax.fori_loop(0, 32, ch_body, jnp.int32(0))
                return 0
            lax.fori_loop(0, NROW, row_body, 0)

            ivec = i0 + _iota16()

            def col_body(k, _):
                ak = cola[k]
                rk = colr[k]
                mask = (ak != 0) & (k < ivec)
                e = _vbroadcast(offv, k) + rk - 1
                val = _vbroadcast(polyv, k).astype(f32) * scale
                plsc.store_scatter(rb2, [_iota16(), V + e], val, mask=mask)
                return 0
            lax.fori_loop(0, V, col_body, 0)

        top_scatter(jnp.float32(1.0))
        pltpu.sync_copy(rb2, out_hbm.at[0, pl.ds(i0, NROW), :])
        top_scatter(jnp.float32(0.0))

        def list_row(l, pos):
            i = i0 + l
            poly_i = _vbroadcast(polyv, i)

            def ch_body(ch, p):
                a = rowsa[l, pl.ds(ch * LANES, LANES)]
                cols = ch * LANES + _iota16()
                u = (a != 0) & (cols > i)
                packed = jnp.full((LANES,), i * 4194304, i32) + cols * 8192 + poly_i
                plsc.store_compressed(plist.at[pl.ds(p, LANES)], packed, mask=u)
                return p + jnp.sum(jnp.where(u, 1, 0))
            return lax.fori_loop(0, 32, ch_body, pos)
        cntw = lax.fori_loop(0, NROW, list_row, jnp.int32(0))
        elo = jnp.max(_vbroadcast(offv, i0))

        ngf = cntw // LANES
        rem = cntw - ngf * LANES
        lane_ids = _iota16()

        def grp_body(g, _):
            word = plist[pl.ds(g * LANES, LANES)]
            iv = word >> 22
            jv = (word >> 13) & 511
            pv = (word & 8191).astype(f32)
            plsc.store_scatter(rb2, [lane_ids, iv], pv)
            plsc.store_scatter(rb2, [lane_ids, jv], pv)
            pltpu.sync_copy(rb2,
                            out_hbm.at[0, pl.ds(V + elo + g * LANES, NROW), :])
            zz = jnp.zeros((LANES,), f32)
            plsc.store_scatter(rb2, [lane_ids, iv], zz)
            plsc.store_scatter(rb2, [lane_ids, jv], zz)
            return 0
        lax.fori_loop(0, ngf, grp_body, 0)

        @pl.when(rem > 0)
        def _tail():
            base = ngf * LANES
            mask = lane_ids < rem
            word = plist[pl.ds(base, LANES)]
            iv = word >> 22
            jv = (word >> 13) & 511
            pv = (word & 8191).astype(f32)
            plsc.store_scatter(rb2, [lane_ids, iv], pv, mask=mask)
            plsc.store_scatter(rb2, [lane_ids, jv], pv, mask=mask)

            def row_out(q, _):
                pltpu.sync_copy(rb2.at[q], out_hbm.at[0, V + elo + base + q, :])
                return 0
            lax.fori_loop(0, rem, row_out, 0)

    pl.run_scoped(
        phase_bc,
        pltpu.VMEM((NROW, V), i32),
        pltpu.VMEM((V, NROW), i32),
        pltpu.VMEM((V, NROW), i32),
        pltpu.VMEM((CAP,), i32),
    )


def _sc_call(A, Pid):
    mesh = plsc.VectorSubcoreMesh(core_axis_name="c", subcore_axis_name="s",
                                  num_cores=2, num_subcores=16)
    fn = functools.partial(
        pl.kernel,
        out_type=jax.ShapeDtypeStruct((1, F, F), f32),
        mesh=mesh,
        compiler_params=pltpu.CompilerParams(use_tc_tiling_on_sc=False,
                                             needs_layout_passes=False),
        scratch_types=[
            pltpu.VMEM_SHARED((V, V), i32),
            pltpu.VMEM_SHARED((V,), i32),
            pltpu.VMEM_SHARED((V,), i32),
            pltpu.VMEM((V,), i32),
            pltpu.VMEM((V,), i32),
            pltpu.VMEM((V,), i32),
            pltpu.VMEM((NROW, F), f32),
        ],
    )(_sc_body)
    return fn(A, Pid)


def kernel(x_prev, c_prev, A, Pid, s_prev):
    x_new, c_new, s_new, a_new = _tc_call(A, x_prev, c_prev, s_prev)
    p_new = _sc_call(A, Pid)
    return (x_new, c_new, a_new, p_new, s_new)


# speedup vs baseline: 3.0359x; 3.0359x over previous
import jax
import jax.numpy as jnp
from jax.experimental import pallas as pl
from jax.experimental.pallas import tpu as pltpu

V = 512
E = 4096
F = V + E
VB = 512
CW = 3 * VB
NB = F // VB
NCB = F // CW
NEB = E // VB
FX = 128
FC = 64


def _one_i(meta_ref, eb):
    cnt = meta_ref[0, :]
    off = meta_ref[1, :]
    poly = meta_ref[2, :]
    e_row = (jax.lax.broadcasted_iota(jnp.int32, (VB, V), 0)
             + eb * VB).astype(jnp.float32)
    one_i = jnp.where(
        (e_row > off[None, :] - 0.5) & (e_row < (off + cnt)[None, :] - 0.5),
        1.0, 0.0)
    pm = jnp.sum(one_i * poly[None, :], axis=1, keepdims=True)
    return one_i, pm, e_row, off


def _build_m(u_ref, rank_ref, meta_ref, eb):
    one_i, pm, e_row, off = _one_i(meta_ref, eb)
    rank_rows = jnp.floor(
        jnp.dot(one_i, rank_ref[...], preferred_element_type=jnp.float32) + 0.5)
    u_rows = rank_rows - jnp.concatenate(
        [jnp.zeros((VB, 1), jnp.float32), rank_rows[:, : V - 1]], axis=1)
    oe = jnp.sum(one_i * off[None, :], axis=1, keepdims=True)
    tgt = e_row[:, 0:1] - oe + 1.0
    one_j = jnp.where((u_rows > 0.5) & (jnp.abs(rank_rows - tgt) < 0.5),
                      1.0, 0.0)
    return one_i + one_j, pm


def _mt_unit(u_st, rank_st, meta_st, m_st, eb):
    m, pm = _build_m(u_st, rank_st, meta_st, eb)
    mt = jnp.transpose(m)
    m_st[pl.ds(eb, 1)] = m[None]
    return mt, mt * jnp.transpose(pm)


def _main_body(a_ref, pid_ref, x_ref, cf_ref, s_ref,
               x_out, c_out, s_out, a_out, p_out,
               m_st, u_st, rank_st, meta_st):
    bi = pl.program_id(0)
    bj = pl.program_id(1)
    zu = jnp.zeros((VB, VB), jnp.float32)

    @pl.when((bi == 0) & (bj == 0))
    def _corner():
        a = a_ref[0]
        row = jax.lax.broadcasted_iota(jnp.int32, (V, V), 0)
        col = jax.lax.broadcasted_iota(jnp.int32, (V, V), 1)
        u = jnp.where((a != 0) & (col > row), 1.0, 0.0).astype(jnp.float32)
        ut = jnp.where(row <= col, 1.0, 0.0).astype(jnp.float32)
        rank = jnp.floor(jnp.dot(u, ut, preferred_element_type=jnp.float32) + 0.5)
        cnt = jnp.sum(u, axis=1)
        sl = jnp.where(row < col, 1.0, 0.0).astype(jnp.float32)
        off = jnp.floor(jnp.sum(cnt[:, None] * sl, axis=0) + 0.5)
        poly = jnp.max(pid_ref[0].astype(jnp.float32), axis=0)
        u_st[...] = u
        rank_st[...] = rank
        meta_st[...] = jnp.concatenate(
            [cnt[None], off[None], poly[None],
             jnp.zeros((5, V), jnp.float32)], axis=0)
        mt0, pt0 = _mt_unit(u_st, rank_st, meta_st, m_st, 0)
        mt1, pt1 = _mt_unit(u_st, rank_st, meta_st, m_st, 1)
        a_out[0] = jnp.concatenate([zu, mt0, mt1], axis=1)
        p_out[0] = jnp.concatenate([zu, pt0, pt1], axis=1)
        x_out[0] = x_ref[0]
        c_out[0] = cf_ref[0]
        s_out[0] = s_ref[0]

    @pl.when((bi == 0) & (bj > 0))
    def _top_band():
        eb = 3 * bj - 1
        mt0, pt0 = _mt_unit(u_st, rank_st, meta_st, m_st, eb)
        mt1, pt1 = _mt_unit(u_st, rank_st, meta_st, m_st, eb + 1)
        mt2, pt2 = _mt_unit(u_st, rank_st, meta_st, m_st, eb + 2)
        a_out[0] = jnp.concatenate([mt0, mt1, mt2], axis=1)
        p_out[0] = jnp.concatenate([pt0, pt1, pt2], axis=1)

    @pl.when((bi > 0) & (bj == 0))
    def _left_band():
        eb = bi - 1
        m = m_st[pl.ds(eb, 1)][0]
        _, pm, _, _ = _one_i(meta_st, eb)
        a_out[0] = jnp.concatenate([m, zu, zu], axis=1)
        p_out[0] = jnp.concatenate([m * pm, zu, zu], axis=1)
        x_out[0] = 0.5 * jnp.dot(m, x_ref[0], preferred_element_type=jnp.float32)
        c_out[0] = 0.5 * jnp.dot(m, cf_ref[0], preferred_element_type=jnp.float32)
        s_out[0] = 0.5 * jnp.dot(m, s_ref[0], preferred_element_type=jnp.float32)

    @pl.when((bi > 0) & (bj > 0))
    def _bulk_zero():
        z = jnp.zeros((VB, CW), jnp.float32)
        a_out[0] = z
        p_out[0] = z


def kernel(x_prev, c_prev, A, Pid, s_prev):
    f32 = jnp.float32
    const = lambda i, j: (0, 0, 0)
    rowblk = lambda i, j: (0, i, 0)
    outs = pl.pallas_call(
        _main_body,
        grid=(NB, NCB),
        in_specs=[
            pl.BlockSpec((1, V, V), const),
            pl.BlockSpec((1, V, V), const),
            pl.BlockSpec((1, V, FX), const),
            pl.BlockSpec((1, V, FC), const),
            pl.BlockSpec((1, V, FX), const),
        ],
        out_specs=[
            pl.BlockSpec((1, VB, FX), rowblk),
            pl.BlockSpec((1, VB, FC), rowblk),
            pl.BlockSpec((1, VB, FX), rowblk),
            pl.BlockSpec((1, VB, CW), lambda i, j: (0, i, j)),
            pl.BlockSpec((1, VB, CW), lambda i, j: (0, i, j)),
        ],
        out_shape=[
            jax.ShapeDtypeStruct((1, F, FX), f32),
            jax.ShapeDtypeStruct((1, F, FC), f32),
            jax.ShapeDtypeStruct((1, F, FX), f32),
            jax.ShapeDtypeStruct((1, F, F), f32),
            jax.ShapeDtypeStruct((1, F, F), f32),
        ],
        scratch_shapes=[
            pltpu.VMEM((NEB, VB, VB), f32),
            pltpu.VMEM((V, V), f32),
            pltpu.VMEM((V, V), f32),
            pltpu.VMEM((8, V), f32),
        ],
    )(A, Pid, x_prev, c_prev, s_prev)
    x_new, c_new, s_new, a_new, p_new = outs
    return (x_new, c_new, a_new, p_new, s_new)


# speedup vs baseline: 3.0449x; 1.0030x over previous
import jax
import jax.numpy as jnp
from jax.experimental import pallas as pl
from jax.experimental.pallas import tpu as pltpu

V = 512
E = 4096
F = V + E
VB = 512
CW = 3 * VB
NB = F // VB
NCB = F // CW
NEB = E // VB
FX = 128
FC = 64


def _one_i(meta_ref, eb):
    cnt = meta_ref[0, :]
    off = meta_ref[1, :]
    poly = meta_ref[2, :]
    e_row = (jax.lax.broadcasted_iota(jnp.int32, (VB, V), 0)
             + eb * VB).astype(jnp.float32)
    one_i = jnp.where(
        (e_row > off[None, :] - 0.5) & (e_row < (off + cnt)[None, :] - 0.5),
        1.0, 0.0)
    pm = jnp.sum(one_i * poly[None, :], axis=1, keepdims=True)
    return one_i, pm, e_row, off


def _build_m(rank_ref, meta_ref, eb):
    one_i, pm, e_row, off = _one_i(meta_ref, eb)
    rank_rows = jnp.floor(
        jnp.dot(one_i, rank_ref[...], preferred_element_type=jnp.float32) + 0.5)
    u_rows = rank_rows - jnp.concatenate(
        [jnp.zeros((VB, 1), jnp.float32), rank_rows[:, : V - 1]], axis=1)
    oe = jnp.sum(one_i * off[None, :], axis=1, keepdims=True)
    tgt = e_row[:, 0:1] - oe + 1.0
    one_j = jnp.where((u_rows > 0.5) & (jnp.abs(rank_rows - tgt) < 0.5),
                      1.0, 0.0)
    return one_i + one_j, pm


def _mt_unit(rank_st, meta_st, m_st, eb):
    m, pm = _build_m(rank_st, meta_st, eb)
    mt = jnp.transpose(m)
    m_st[pl.ds(eb, 1)] = m[None]
    return mt, mt * jnp.transpose(pm)


def _main_body(a_ref, pid_ref, x_ref, cf_ref, s_ref,
               x_out, c_out, s_out, a_out, p_out,
               m_st, rank_st, meta_st):
    bi = pl.program_id(0)
    bj = pl.program_id(1)
    zu = jnp.zeros((VB, VB), jnp.float32)

    @pl.when((bi == 0) & (bj == 0))
    def _corner():
        a = a_ref[0]
        row = jax.lax.broadcasted_iota(jnp.int32, (V, V), 0)
        col = jax.lax.broadcasted_iota(jnp.int32, (V, V), 1)
        u = jnp.where((a != 0) & (col > row), 1.0, 0.0).astype(jnp.float32)
        ut = jnp.where(row <= col, 1.0, 0.0).astype(jnp.float32)
        rank = jnp.floor(jnp.dot(u, ut, preferred_element_type=jnp.float32) + 0.5)
        cnt = jnp.sum(u, axis=1)
        sl = jnp.where(row < col, 1.0, 0.0).astype(jnp.float32)
        off = jnp.floor(jnp.sum(cnt[:, None] * sl, axis=0) + 0.5)
        poly = jnp.max(pid_ref[0].astype(jnp.float32), axis=0)
        rank_st[...] = rank
        meta_st[...] = jnp.concatenate(
            [cnt[None], off[None], poly[None],
             jnp.zeros((5, V), jnp.float32)], axis=0)
        mt0, pt0 = _mt_unit(rank_st, meta_st, m_st, 0)
        mt1, pt1 = _mt_unit(rank_st, meta_st, m_st, 1)
        a_out[0] = jnp.concatenate([zu, mt0, mt1], axis=1)
        p_out[0] = jnp.concatenate([zu, pt0, pt1], axis=1)
        x_out[0] = x_ref[0]
        c_out[0] = cf_ref[0]
        s_out[0] = s_ref[0]

    @pl.when((bi == 0) & (bj > 0))
    def _top_band():
        eb = 3 * bj - 1
        mt0, pt0 = _mt_unit(rank_st, meta_st, m_st, eb)
        mt1, pt1 = _mt_unit(rank_st, meta_st, m_st, eb + 1)
        mt2, pt2 = _mt_unit(rank_st, meta_st, m_st, eb + 2)
        a_out[0] = jnp.concatenate([mt0, mt1, mt2], axis=1)
        p_out[0] = jnp.concatenate([pt0, pt1, pt2], axis=1)

    @pl.when((bi > 0) & (bj == 0))
    def _left_band():
        eb = bi - 1
        m = m_st[pl.ds(eb, 1)][0]
        _, pm, _, _ = _one_i(meta_st, eb)
        a_out[0] = jnp.concatenate([m, zu, zu], axis=1)
        p_out[0] = jnp.concatenate([m * pm, zu, zu], axis=1)
        x_out[0] = 0.5 * jnp.dot(m, x_ref[0], preferred_element_type=jnp.float32)
        c_out[0] = 0.5 * jnp.dot(m, cf_ref[0], preferred_element_type=jnp.float32)
        s_out[0] = 0.5 * jnp.dot(m, s_ref[0], preferred_element_type=jnp.float32)

    @pl.when((bi > 0) & (bj > 0))
    def _bulk_zero():
        z = jnp.zeros((VB, CW), jnp.float32)
        a_out[0] = z
        p_out[0] = z


def kernel(x_prev, c_prev, A, Pid, s_prev):
    f32 = jnp.float32
    const = lambda i, j: (0, 0, 0)
    rowblk = lambda i, j: (0, i, 0)
    outs = pl.pallas_call(
        _main_body,
        grid=(NB, NCB),
        in_specs=[
            pl.BlockSpec((1, V, V), const),
            pl.BlockSpec((1, V, V), const),
            pl.BlockSpec((1, V, FX), const),
            pl.BlockSpec((1, V, FC), const),
            pl.BlockSpec((1, V, FX), const),
        ],
        out_specs=[
            pl.BlockSpec((1, VB, FX), rowblk),
            pl.BlockSpec((1, VB, FC), rowblk),
            pl.BlockSpec((1, VB, FX), rowblk),
            pl.BlockSpec((1, VB, CW), lambda i, j: (0, i, j)),
            pl.BlockSpec((1, VB, CW), lambda i, j: (0, i, j)),
        ],
        out_shape=[
            jax.ShapeDtypeStruct((1, F, FX), f32),
            jax.ShapeDtypeStruct((1, F, FC), f32),
            jax.ShapeDtypeStruct((1, F, FX), f32),
            jax.ShapeDtypeStruct((1, F, F), f32),
            jax.ShapeDtypeStruct((1, F, F), f32),
        ],
        scratch_shapes=[
            pltpu.VMEM((NEB, VB, VB), f32),
            pltpu.VMEM((V, V), f32),
            pltpu.VMEM((8, V), f32),
        ],
    )(A, Pid, x_prev, c_prev, s_prev)
    x_new, c_new, s_new, a_new, p_new = outs
    return (x_new, c_new, a_new, p_new, s_new)
